# Initial kernel scaffold; baseline (speedup 1.0000x reference)
import jax
import jax.numpy as jnp
from jax.experimental import pallas as pl


def kernel(x, edge_attr, W0, b0, We, Wl, bl, ln_g, ln_b, vemb, vW1, vb1, vg1, vbt1, vW2, vb2, vg2, vbt2, Wo, bo, edge_index, batch):
    raise NotImplementedError("write your pallas kernel here")


# speedup vs baseline: 2.0506x; 2.0506x over previous
import functools

import jax
import jax.numpy as jnp
from jax import lax
from jax.experimental import pallas as pl
from jax.experimental.pallas import tpu as pltpu
from jax.experimental.pallas import tpu_sc as plsc

_N = 10000
_E = 320000
_D = 128
_DE = 16
_G = 64
_L = 3
_H2 = 256

_NC = 2
_NS = 16
_NW = _NC * _NS
_C = 128
_EPW = 10112
_E_PAD = _NW * _EPW
_NCHUNK = _EPW // _C
_N_PAD = 10112
_RPT = _N // _NS

_BN = 2000
_NB = _N // _BN
_BE = _E_PAD // 32

_HI = lax.Precision.HIGHEST
_F32 = jnp.float32


def _dot(a, b):
    return jnp.dot(a, b, precision=_HI, preferred_element_type=_F32)


def _em_body(ea_ref, we_ref, out_ref):
    out_ref[0] = _dot(ea_ref[:], we_ref[0])


_em_call = pl.pallas_call(
    _em_body,
    grid=(_L, _E_PAD // _BE),
    in_specs=[
        pl.BlockSpec((_BE, _DE), lambda l, i: (i, 0)),
        pl.BlockSpec((1, _DE, _D), lambda l, i: (l, 0, 0)),
    ],
    out_specs=pl.BlockSpec((1, _BE, _D), lambda l, i: (l, i, 0)),
    out_shape=jax.ShapeDtypeStruct((_L, _E_PAD, _D), _F32),
)


def _onehot_t(batch_ref):
    bb = batch_ref[0]
    return (jnp.broadcast_to(bb, (_G, _BN))
            == lax.broadcasted_iota(jnp.int32, (_G, _BN), 0)).astype(_F32)


def _init_body(x_ref, w0_ref, b0_ref, vemb_ref, batch_ref, hin_ref, pool_ref):
    h = _dot(x_ref[:], w0_ref[:]) + b0_ref[:] + vemb_ref[:]
    hin_ref[:] = h
    oht = _onehot_t(batch_ref)
    pool_blk = lax.dot_general(oht, h, (((1,), (0,)), ((), ())),
                               precision=_HI, preferred_element_type=_F32)

    @pl.when(pl.program_id(0) == 0)
    def _zero():
        pool_ref[:] = jnp.zeros_like(pool_ref)

    pool_ref[:] += pool_blk


_init_call = pl.pallas_call(
    _init_body,
    grid=(_NB,),
    in_specs=[
        pl.BlockSpec((_BN, _D), lambda i: (i, 0)),
        pl.BlockSpec((_D, _D), lambda i: (0, 0)),
        pl.BlockSpec((1, _D), lambda i: (0, 0)),
        pl.BlockSpec((1, _D), lambda i: (0, 0)),
        pl.BlockSpec((1, 1, _BN), lambda i: (i, 0, 0)),
    ],
    out_specs=[
        pl.BlockSpec((_BN, _D), lambda i: (i, 0)),
        pl.BlockSpec((_G, _D), lambda i: (0, 0)),
    ],
    out_shape=[
        jax.ShapeDtypeStruct((_N, _D), _F32),
        jax.ShapeDtypeStruct((_G, _D), _F32),
    ],
)


def _vn_body(pool_ref, v_ref, w1_ref, b1_ref, g1_ref, t1_ref,
             w2_ref, b2_ref, g2_ref, t2_ref, out_ref):
    vt = pool_ref[:] + v_ref[:]
    z = _dot(vt, w1_ref[:]) + b1_ref[:]
    mu = jnp.mean(z, axis=0, keepdims=True)
    var = jnp.mean((z - mu) ** 2, axis=0, keepdims=True)
    z = jnp.maximum(g1_ref[:] * (z - mu) / jnp.sqrt(var + 1e-5) + t1_ref[:], 0.0)
    z2 = _dot(z, w2_ref[:]) + b2_ref[:]
    mu2 = jnp.mean(z2, axis=0, keepdims=True)
    var2 = jnp.mean((z2 - mu2) ** 2, axis=0, keepdims=True)
    out_ref[:] = jnp.maximum(
        g2_ref[:] * (z2 - mu2) / jnp.sqrt(var2 + 1e-5) + t2_ref[:], 0.0)


_vn_call = pl.pallas_call(
    _vn_body,
    out_shape=jax.ShapeDtypeStruct((_G, _D), _F32),
)


def _pp_body(part_ref, wl_ref, bl_ref, lg_ref, lb_ref, v_ref, batch_ref,
             hin_ref, pool_ref):
    aggb = part_ref[0] + part_ref[1]
    hl = _dot(aggb, wl_ref[:]) + bl_ref[:]
    mu = jnp.mean(hl, axis=-1, keepdims=True)
    var = jnp.mean((hl - mu) ** 2, axis=-1, keepdims=True)
    hl = lg_ref[:] * (hl - mu) / jnp.sqrt(var + 1e-5) + lb_ref[:]
    oht = _onehot_t(batch_ref)
    hin = hl + lax.dot_general(oht, v_ref[:], (((0,), (0,)), ((), ())),
                               precision=_HI, preferred_element_type=_F32)
    hin_ref[:] = hin
    pool_blk = lax.dot_general(oht, hin, (((1,), (0,)), ((), ())),
                               precision=_HI, preferred_element_type=_F32)

    @pl.when(pl.program_id(0) == 0)
    def _zero():
        pool_ref[:] = jnp.zeros_like(pool_ref)

    pool_ref[:] += pool_blk


_pp_call = pl.pallas_call(
    _pp_body,
    grid=(_NB,),
    in_specs=[
        pl.BlockSpec((2, _BN, _D), lambda i: (0, i, 0)),
        pl.BlockSpec((_D, _D), lambda i: (0, 0)),
        pl.BlockSpec((1, _D), lambda i: (0, 0)),
        pl.BlockSpec((1, _D), lambda i: (0, 0)),
        pl.BlockSpec((1, _D), lambda i: (0, 0)),
        pl.BlockSpec((_G, _D), lambda i: (0, 0)),
        pl.BlockSpec((1, 1, _BN), lambda i: (i, 0, 0)),
    ],
    out_specs=[
        pl.BlockSpec((_BN, _D), lambda i: (i, 0)),
        pl.BlockSpec((_G, _D), lambda i: (0, 0)),
    ],
    out_shape=[
        jax.ShapeDtypeStruct((_N, _D), _F32),
        jax.ShapeDtypeStruct((_G, _D), _F32),
    ],
)


def _final_body(part_ref, wl_ref, bl_ref, lg_ref, lb_ref, wo_ref, bo_ref,
                out_ref):
    aggb = part_ref[0] + part_ref[1]
    hl = _dot(aggb, wl_ref[:]) + bl_ref[:]
    mu = jnp.mean(hl, axis=-1, keepdims=True)
    var = jnp.mean((hl - mu) ** 2, axis=-1, keepdims=True)
    hl = lg_ref[:] * (hl - mu) / jnp.sqrt(var + 1e-5) + lb_ref[:]
    out_ref[:] = jnp.maximum(_dot(hl, wo_ref[:]) + bo_ref[:], 0.0)


_final_call = pl.pallas_call(
    _final_body,
    grid=(_NB,),
    in_specs=[
        pl.BlockSpec((2, _BN, _D), lambda i: (0, i, 0)),
        pl.BlockSpec((_D, _D), lambda i: (0, 0)),
        pl.BlockSpec((1, _D), lambda i: (0, 0)),
        pl.BlockSpec((1, _D), lambda i: (0, 0)),
        pl.BlockSpec((1, _D), lambda i: (0, 0)),
        pl.BlockSpec((_D, _D), lambda i: (0, 0)),
        pl.BlockSpec((1, _D), lambda i: (0, 0)),
    ],
    out_specs=pl.BlockSpec((_BN, _D), lambda i: (i, 0)),
    out_shape=jax.ShapeDtypeStruct((_N, _D), _F32),
)


def _make_mp(layer):
    @functools.partial(
        pl.kernel,
        out_type=jax.ShapeDtypeStruct((2 * _N, _D), _F32),
        mesh=plsc.VectorSubcoreMesh(core_axis_name="c", subcore_axis_name="s"),
        scratch_types=[
            pltpu.VMEM((_C,), jnp.int32),
            pltpu.VMEM((_C,), jnp.int32),
            pltpu.VMEM((_C, _D), _F32),
            pltpu.VMEM((_C, _D), _F32),
            pltpu.VMEM_SHARED((_N_PAD, _D), _F32),
            pltpu.SemaphoreType.DMA,
        ],
    )
    def mp(hin_hbm, em_hbm, src_hbm, dst_hbm, zeros_hbm, out_hbm,
           sidx, didx, rows, emb, agg, sem):
        c = lax.axis_index("c")
        s = lax.axis_index("s")
        wid = c * _NS + s

        pltpu.sync_copy(zeros_hbm, agg.at[pl.ds(s * (_N_PAD // _NS),
                                                _N_PAD // _NS)])
        plsc.subcore_barrier()

        def chunk(g, carry):
            off = wid * _EPW + g * _C
            pltpu.sync_copy(src_hbm.at[pl.ds(off, _C)], sidx)
            pltpu.sync_copy(dst_hbm.at[pl.ds(off, _C)], didx)
            pltpu.sync_copy(em_hbm.at[layer, pl.ds(off, _C), :], emb)
            pltpu.async_copy(hin_hbm.at[sidx], rows, sem).wait()

            def edge(e, c2):
                for q in range(8):
                    sl = pl.ds(q * 16, 16)
                    rows[e, sl] = jnp.maximum(rows[e, sl] + emb[e, sl], 0.0)
                return c2

            lax.fori_loop(0, _C, edge, 0)
            pltpu.sync_copy(rows, agg.at[didx], add=True)
            return carry

        lax.fori_loop(0, _NCHUNK, chunk, 0)

        plsc.subcore_barrier()

        @pl.when(s < _NS - 1)
        def _wb_body():
            pltpu.sync_copy(agg.at[pl.ds(s * 624, 624)],
                            out_hbm.at[pl.ds(c * _N + s * 624, 624), :])

        @pl.when(s == _NS - 1)
        def _wb_tail():
            pltpu.sync_copy(agg.at[pl.ds((_NS - 1) * 624, 640)],
                            out_hbm.at[pl.ds(c * _N + (_NS - 1) * 624, 640), :])

    return mp


_mp_calls = [_make_mp(l) for l in range(_L)]


def kernel(x, edge_attr, W0, b0, We, Wl, bl, ln_g, ln_b, vemb, vW1, vb1,
           vg1, vbt1, vW2, vb2, vg2, vbt2, Wo, bo, edge_index, batch):
    src = edge_index[0]
    dst = edge_index[1]
    pad = _E_PAD - _E
    src_p = jnp.concatenate([src, jnp.zeros((pad,), jnp.int32)])
    dst_p = jnp.concatenate([dst, jnp.full((pad,), _N, jnp.int32)])
    ea_p = jnp.concatenate([edge_attr, jnp.zeros((pad, _DE), _F32)], axis=0)
    batch3 = batch.reshape(_NB, 1, _BN)
    b0r = b0.reshape(1, _D)

    zeros2d = jnp.zeros((_N_PAD // _NS, _D), _F32)

    em = _em_call(ea_p, We)
    hin, pool = _init_call(x, W0, b0r, vemb, batch3)
    v = jnp.broadcast_to(vemb, (_G, _D))
    out = None
    for l in range(_L):
        part = _mp_calls[l](hin, em, src_p, dst_p, zeros2d).reshape(2, _N, _D)
        if l < _L - 1:
            v = _vn_call(pool, v, vW1[l], vb1[l].reshape(1, _H2),
                         vg1[l].reshape(1, _H2), vbt1[l].reshape(1, _H2),
                         vW2[l], vb2[l].reshape(1, _D), vg2[l].reshape(1, _D),
                         vbt2[l].reshape(1, _D))
            hin, pool = _pp_call(part, Wl[l], bl[l].reshape(1, _D),
                                 ln_g[l].reshape(1, _D), ln_b[l].reshape(1, _D),
                                 v, batch3)
        else:
            out = _final_call(part, Wl[l], bl[l].reshape(1, _D),
                              ln_g[l].reshape(1, _D), ln_b[l].reshape(1, _D),
                              Wo, bo.reshape(1, _D))
    return out


# speedup vs baseline: 2.5887x; 1.2624x over previous
import functools

import jax
import jax.numpy as jnp
from jax import lax
from jax.experimental import pallas as pl
from jax.experimental.pallas import tpu as pltpu
from jax.experimental.pallas import tpu_sc as plsc

_N = 10000
_E = 320000
_D = 128
_DE = 16
_G = 64
_L = 3
_H2 = 256

_NC = 2
_NS = 16
_NW = _NC * _NS
_C = 48
_NBUF = 4
_NCHUNK = 212
_EPW = _C * _NCHUNK
_E_PAD = _NW * _EPW
_N_PAD = 10112
_RPT = _N // _NS

_BN = 2000
_NB = _N // _BN
_BE = _E_PAD // 32

_HI = lax.Precision.HIGHEST
_F32 = jnp.float32


def _dot(a, b):
    return jnp.dot(a, b, precision=_HI, preferred_element_type=_F32)


def _em_body(ea_ref, we_ref, out_ref):
    out_ref[0] = _dot(ea_ref[:], we_ref[0])


_em_call = pl.pallas_call(
    _em_body,
    grid=(_L, _E_PAD // _BE),
    in_specs=[
        pl.BlockSpec((_BE, _DE), lambda l, i: (i, 0)),
        pl.BlockSpec((1, _DE, _D), lambda l, i: (l, 0, 0)),
    ],
    out_specs=pl.BlockSpec((1, _BE, _D), lambda l, i: (l, i, 0)),
    out_shape=jax.ShapeDtypeStruct((_L, _E_PAD, _D), _F32),
)


def _onehot_t(batch_ref):
    bb = batch_ref[0]
    return (jnp.broadcast_to(bb, (_G, _BN))
            == lax.broadcasted_iota(jnp.int32, (_G, _BN), 0)).astype(_F32)


def _init_body(x_ref, w0_ref, b0_ref, vemb_ref, batch_ref, hin_ref, pool_ref):
    h = _dot(x_ref[:], w0_ref[:]) + b0_ref[:] + vemb_ref[:]
    hin_ref[:] = h
    oht = _onehot_t(batch_ref)
    pool_blk = lax.dot_general(oht, h, (((1,), (0,)), ((), ())),
                               precision=_HI, preferred_element_type=_F32)

    @pl.when(pl.program_id(0) == 0)
    def _zero():
        pool_ref[:] = jnp.zeros_like(pool_ref)

    pool_ref[:] += pool_blk


_init_call = pl.pallas_call(
    _init_body,
    grid=(_NB,),
    in_specs=[
        pl.BlockSpec((_BN, _D), lambda i: (i, 0)),
        pl.BlockSpec((_D, _D), lambda i: (0, 0)),
        pl.BlockSpec((1, _D), lambda i: (0, 0)),
        pl.BlockSpec((1, _D), lambda i: (0, 0)),
        pl.BlockSpec((1, 1, _BN), lambda i: (i, 0, 0)),
    ],
    out_specs=[
        pl.BlockSpec((_BN, _D), lambda i: (i, 0)),
        pl.BlockSpec((_G, _D), lambda i: (0, 0)),
    ],
    out_shape=[
        jax.ShapeDtypeStruct((_N, _D), _F32),
        jax.ShapeDtypeStruct((_G, _D), _F32),
    ],
)


def _vn_body(pool_ref, v_ref, w1_ref, b1_ref, g1_ref, t1_ref,
             w2_ref, b2_ref, g2_ref, t2_ref, out_ref):
    vt = pool_ref[:] + v_ref[:]
    z = _dot(vt, w1_ref[:]) + b1_ref[:]
    mu = jnp.mean(z, axis=0, keepdims=True)
    var = jnp.mean((z - mu) ** 2, axis=0, keepdims=True)
    z = jnp.maximum(g1_ref[:] * (z - mu) / jnp.sqrt(var + 1e-5) + t1_ref[:], 0.0)
    z2 = _dot(z, w2_ref[:]) + b2_ref[:]
    mu2 = jnp.mean(z2, axis=0, keepdims=True)
    var2 = jnp.mean((z2 - mu2) ** 2, axis=0, keepdims=True)
    out_ref[:] = jnp.maximum(
        g2_ref[:] * (z2 - mu2) / jnp.sqrt(var2 + 1e-5) + t2_ref[:], 0.0)


_vn_call = pl.pallas_call(
    _vn_body,
    out_shape=jax.ShapeDtypeStruct((_G, _D), _F32),
)


def _pp_body(part_ref, wl_ref, bl_ref, lg_ref, lb_ref, v_ref, batch_ref,
             hin_ref, pool_ref):
    aggb = part_ref[0] + part_ref[1]
    hl = _dot(aggb, wl_ref[:]) + bl_ref[:]
    mu = jnp.mean(hl, axis=-1, keepdims=True)
    var = jnp.mean((hl - mu) ** 2, axis=-1, keepdims=True)
    hl = lg_ref[:] * (hl - mu) / jnp.sqrt(var + 1e-5) + lb_ref[:]
    oht = _onehot_t(batch_ref)
    hin = hl + lax.dot_general(oht, v_ref[:], (((0,), (0,)), ((), ())),
                               precision=_HI, preferred_element_type=_F32)
    hin_ref[:] = hin
    pool_blk = lax.dot_general(oht, hin, (((1,), (0,)), ((), ())),
                               precision=_HI, preferred_element_type=_F32)

    @pl.when(pl.program_id(0) == 0)
    def _zero():
        pool_ref[:] = jnp.zeros_like(pool_ref)

    pool_ref[:] += pool_blk


_pp_call = pl.pallas_call(
    _pp_body,
    grid=(_NB,),
    in_specs=[
        pl.BlockSpec((2, _BN, _D), lambda i: (0, i, 0)),
        pl.BlockSpec((_D, _D), lambda i: (0, 0)),
        pl.BlockSpec((1, _D), lambda i: (0, 0)),
        pl.BlockSpec((1, _D), lambda i: (0, 0)),
        pl.BlockSpec((1, _D), lambda i: (0, 0)),
        pl.BlockSpec((_G, _D), lambda i: (0, 0)),
        pl.BlockSpec((1, 1, _BN), lambda i: (i, 0, 0)),
    ],
    out_specs=[
        pl.BlockSpec((_BN, _D), lambda i: (i, 0)),
        pl.BlockSpec((_G, _D), lambda i: (0, 0)),
    ],
    out_shape=[
        jax.ShapeDtypeStruct((_N, _D), _F32),
        jax.ShapeDtypeStruct((_G, _D), _F32),
    ],
)


def _final_body(part_ref, wl_ref, bl_ref, lg_ref, lb_ref, wo_ref, bo_ref,
                out_ref):
    aggb = part_ref[0] + part_ref[1]
    hl = _dot(aggb, wl_ref[:]) + bl_ref[:]
    mu = jnp.mean(hl, axis=-1, keepdims=True)
    var = jnp.mean((hl - mu) ** 2, axis=-1, keepdims=True)
    hl = lg_ref[:] * (hl - mu) / jnp.sqrt(var + 1e-5) + lb_ref[:]
    out_ref[:] = jnp.maximum(_dot(hl, wo_ref[:]) + bo_ref[:], 0.0)


_final_call = pl.pallas_call(
    _final_body,
    grid=(_NB,),
    in_specs=[
        pl.BlockSpec((2, _BN, _D), lambda i: (0, i, 0)),
        pl.BlockSpec((_D, _D), lambda i: (0, 0)),
        pl.BlockSpec((1, _D), lambda i: (0, 0)),
        pl.BlockSpec((1, _D), lambda i: (0, 0)),
        pl.BlockSpec((1, _D), lambda i: (0, 0)),
        pl.BlockSpec((_D, _D), lambda i: (0, 0)),
        pl.BlockSpec((1, _D), lambda i: (0, 0)),
    ],
    out_specs=pl.BlockSpec((_BN, _D), lambda i: (i, 0)),
    out_shape=jax.ShapeDtypeStruct((_N, _D), _F32),
)


def _make_mp(layer):
    scratch = ([pltpu.VMEM((_C,), jnp.int32)] * _NBUF
               + [pltpu.VMEM((_C,), jnp.int32)] * _NBUF
               + [pltpu.VMEM((_C, _D), _F32)] * _NBUF
               + [pltpu.VMEM((_C, _D), _F32)] * _NBUF
               + [pltpu.VMEM_SHARED((_N_PAD, _D), _F32)]
               + [pltpu.SemaphoreType.DMA] * (3 * _NBUF))

    @functools.partial(
        pl.kernel,
        out_type=jax.ShapeDtypeStruct((2 * _N, _D), _F32),
        mesh=plsc.VectorSubcoreMesh(core_axis_name="c", subcore_axis_name="s"),
        scratch_types=scratch,
    )
    def mp(hin_hbm, em_hbm, src_hbm, dst_hbm, zeros_hbm, out_hbm, *sc):
        sidx = sc[0:_NBUF]
        didx = sc[_NBUF:2 * _NBUF]
        rows = sc[2 * _NBUF:3 * _NBUF]
        emb = sc[3 * _NBUF:4 * _NBUF]
        agg = sc[4 * _NBUF]
        semA = sc[4 * _NBUF + 1:4 * _NBUF + 1 + _NBUF]
        semG = sc[4 * _NBUF + 1 + _NBUF:4 * _NBUF + 1 + 2 * _NBUF]
        semS = sc[4 * _NBUF + 1 + 2 * _NBUF:4 * _NBUF + 1 + 3 * _NBUF]

        c = lax.axis_index("c")
        s = lax.axis_index("s")
        wid = c * _NS + s

        def startA(g, j):
            off = wid * _EPW + g * _C
            pltpu.async_copy(src_hbm.at[pl.ds(off, _C)], sidx[j], semA[j])
            pltpu.async_copy(dst_hbm.at[pl.ds(off, _C)], didx[j], semA[j])
            pltpu.async_copy(em_hbm.at[layer, pl.ds(off, _C), :], emb[j],
                             semA[j])

        def waitA(j):
            pltpu.make_async_copy(src_hbm.at[pl.ds(0, _C)], sidx[j],
                                  semA[j]).wait()
            pltpu.make_async_copy(dst_hbm.at[pl.ds(0, _C)], didx[j],
                                  semA[j]).wait()
            pltpu.make_async_copy(em_hbm.at[layer, pl.ds(0, _C), :], emb[j],
                                  semA[j]).wait()

        def startG(j):
            pltpu.async_copy(hin_hbm.at[sidx[j]], rows[j], semG[j])

        def waitG(j):
            pltpu.make_async_copy(hin_hbm.at[sidx[j]], rows[j], semG[j]).wait()

        def compute(j):
            r, m = rows[j], emb[j]

            def edge(e, c2):
                for q in range(8):
                    sl = pl.ds(q * 16, 16)
                    r[e, sl] = jnp.maximum(r[e, sl] + m[e, sl], 0.0)
                return c2

            lax.fori_loop(0, _C, edge, 0)

        def startS(j):
            pltpu.async_copy(rows[j], agg.at[didx[j]], semS[j], add=True)

        def waitS(j):
            pltpu.make_async_copy(rows[j], agg.at[didx[j]], semS[j]).wait()

        pltpu.sync_copy(zeros_hbm, agg.at[pl.ds(s * (_N_PAD // _NS),
                                                _N_PAD // _NS)])
        plsc.subcore_barrier()

        startA(0, 0)
        startA(1, 1)
        waitA(0)
        startG(0)

        def quad(p, carry):
            for j in range(_NBUF):
                g = _NBUF * p + j
                j1 = (j + 1) % _NBUF
                j2 = (j + 2) % _NBUF

                @pl.when(g >= 2)
                def _ws(j2=j2):
                    waitS(j2)

                @pl.when(g + 2 < _NCHUNK)
                def _sa(g=g, j2=j2):
                    startA(g + 2, j2)

                @pl.when(g + 1 < _NCHUNK)
                def _sg(j1=j1):
                    waitA(j1)
                    startG(j1)

                waitG(j)
                compute(j)
                startS(j)
            return carry

        lax.fori_loop(0, _NCHUNK // _NBUF, quad, 0)
        waitS((_NCHUNK - 2) % _NBUF)
        waitS((_NCHUNK - 1) % _NBUF)

        plsc.subcore_barrier()

        @pl.when(s < _NS - 1)
        def _wb_body():
            pltpu.sync_copy(agg.at[pl.ds(s * 624, 624)],
                            out_hbm.at[pl.ds(c * _N + s * 624, 624), :])

        @pl.when(s == _NS - 1)
        def _wb_tail():
            pltpu.sync_copy(agg.at[pl.ds((_NS - 1) * 624, 640)],
                            out_hbm.at[pl.ds(c * _N + (_NS - 1) * 624, 640), :])

    return mp


_mp_calls = [_make_mp(l) for l in range(_L)]


def kernel(x, edge_attr, W0, b0, We, Wl, bl, ln_g, ln_b, vemb, vW1, vb1,
           vg1, vbt1, vW2, vb2, vg2, vbt2, Wo, bo, edge_index, batch):
    src = edge_index[0]
    dst = edge_index[1]
    pad = _E_PAD - _E
    src_p = jnp.concatenate([src, jnp.zeros((pad,), jnp.int32)])
    dst_p = jnp.concatenate([dst, jnp.full((pad,), _N, jnp.int32)])
    ea_p = jnp.concatenate([edge_attr, jnp.zeros((pad, _DE), _F32)], axis=0)
    batch3 = batch.reshape(_NB, 1, _BN)
    b0r = b0.reshape(1, _D)

    zeros2d = jnp.zeros((_N_PAD // _NS, _D), _F32)

    em = _em_call(ea_p, We)
    hin, pool = _init_call(x, W0, b0r, vemb, batch3)
    v = jnp.broadcast_to(vemb, (_G, _D))
    out = None
    for l in range(_L):
        part = _mp_calls[l](hin, em, src_p, dst_p, zeros2d).reshape(2, _N, _D)
        if l < _L - 1:
            v = _vn_call(pool, v, vW1[l], vb1[l].reshape(1, _H2),
                         vg1[l].reshape(1, _H2), vbt1[l].reshape(1, _H2),
                         vW2[l], vb2[l].reshape(1, _D), vg2[l].reshape(1, _D),
                         vbt2[l].reshape(1, _D))
            hin, pool = _pp_call(part, Wl[l], bl[l].reshape(1, _D),
                                 ln_g[l].reshape(1, _D), ln_b[l].reshape(1, _D),
                                 v, batch3)
        else:
            out = _final_call(part, Wl[l], bl[l].reshape(1, _D),
                              ln_g[l].reshape(1, _D), ln_b[l].reshape(1, _D),
                              Wo, bo.reshape(1, _D))
    return out


# speedup vs baseline: 2.8421x; 1.0979x over previous
import functools

import jax
import jax.numpy as jnp
from jax import lax
from jax.experimental import pallas as pl
from jax.experimental.pallas import tpu as pltpu
from jax.experimental.pallas import tpu_sc as plsc

_N = 10000
_E = 320000
_D = 128
_DE = 16
_G = 64
_L = 3
_H2 = 256

_NC = 2
_NS = 16
_NW = _NC * _NS
_C = 48
_NBUF = 4
_T0 = 292
_T1 = 132
_NCHUNK = _T0 + _T1
_E_PAD = _NS * _NCHUNK * _C
_N_PAD = 10112
_RPT = _N // _NS

_BN = 2000
_NB = _N // _BN
_BE = _E_PAD // 32

_HI = lax.Precision.HIGHEST
_F32 = jnp.float32


def _dot(a, b):
    return jnp.dot(a, b, precision=_HI, preferred_element_type=_F32)


def _em_body(ea_ref, we_ref, out_ref):
    out_ref[:] = _dot(ea_ref[:], we_ref[:])


_em_call = pl.pallas_call(
    _em_body,
    grid=(_E_PAD // _BE,),
    in_specs=[
        pl.BlockSpec((_BE, _DE), lambda i: (i, 0)),
        pl.BlockSpec((_DE, _D), lambda i: (0, 0)),
    ],
    out_specs=pl.BlockSpec((_BE, _D), lambda i: (i, 0)),
    out_shape=jax.ShapeDtypeStruct((_E_PAD, _D), _F32),
)


def _onehot_t(batch_ref):
    bb = batch_ref[0]
    return (jnp.broadcast_to(bb, (_G, _BN))
            == lax.broadcasted_iota(jnp.int32, (_G, _BN), 0)).astype(_F32)


def _init_body(x_ref, w0_ref, b0_ref, vemb_ref, batch_ref, hin_ref, pool_ref):
    h = _dot(x_ref[:], w0_ref[:]) + b0_ref[:] + vemb_ref[:]
    hin_ref[:] = h
    oht = _onehot_t(batch_ref)
    pool_blk = lax.dot_general(oht, h, (((1,), (0,)), ((), ())),
                               precision=_HI, preferred_element_type=_F32)

    @pl.when(pl.program_id(0) == 0)
    def _zero():
        pool_ref[:] = jnp.zeros_like(pool_ref)

    pool_ref[:] += pool_blk


_init_call = pl.pallas_call(
    _init_body,
    grid=(_NB,),
    in_specs=[
        pl.BlockSpec((_BN, _D), lambda i: (i, 0)),
        pl.BlockSpec((_D, _D), lambda i: (0, 0)),
        pl.BlockSpec((1, _D), lambda i: (0, 0)),
        pl.BlockSpec((1, _D), lambda i: (0, 0)),
        pl.BlockSpec((1, 1, _BN), lambda i: (i, 0, 0)),
    ],
    out_specs=[
        pl.BlockSpec((_BN, _D), lambda i: (i, 0)),
        pl.BlockSpec((_G, _D), lambda i: (0, 0)),
    ],
    out_shape=[
        jax.ShapeDtypeStruct((_N, _D), _F32),
        jax.ShapeDtypeStruct((_G, _D), _F32),
    ],
)


def _vn_body(pool_ref, v_ref, w1_ref, b1_ref, g1_ref, t1_ref,
             w2_ref, b2_ref, g2_ref, t2_ref, out_ref):
    vt = pool_ref[:] + v_ref[:]
    z = _dot(vt, w1_ref[:]) + b1_ref[:]
    mu = jnp.mean(z, axis=0, keepdims=True)
    var = jnp.mean((z - mu) ** 2, axis=0, keepdims=True)
    z = jnp.maximum(g1_ref[:] * (z - mu) / jnp.sqrt(var + 1e-5) + t1_ref[:], 0.0)
    z2 = _dot(z, w2_ref[:]) + b2_ref[:]
    mu2 = jnp.mean(z2, axis=0, keepdims=True)
    var2 = jnp.mean((z2 - mu2) ** 2, axis=0, keepdims=True)
    out_ref[:] = jnp.maximum(
        g2_ref[:] * (z2 - mu2) / jnp.sqrt(var2 + 1e-5) + t2_ref[:], 0.0)


_vn_call = pl.pallas_call(
    _vn_body,
    out_shape=jax.ShapeDtypeStruct((_G, _D), _F32),
)


def _pp_body(part_ref, wl_ref, bl_ref, lg_ref, lb_ref, v_ref, batch_ref,
             hin_ref, pool_ref):
    aggb = part_ref[0] + part_ref[1]
    hl = _dot(aggb, wl_ref[:]) + bl_ref[:]
    mu = jnp.mean(hl, axis=-1, keepdims=True)
    var = jnp.mean((hl - mu) ** 2, axis=-1, keepdims=True)
    hl = lg_ref[:] * (hl - mu) / jnp.sqrt(var + 1e-5) + lb_ref[:]
    oht = _onehot_t(batch_ref)
    hin = hl + lax.dot_general(oht, v_ref[:], (((0,), (0,)), ((), ())),
                               precision=_HI, preferred_element_type=_F32)
    hin_ref[:] = hin
    pool_blk = lax.dot_general(oht, hin, (((1,), (0,)), ((), ())),
                               precision=_HI, preferred_element_type=_F32)

    @pl.when(pl.program_id(0) == 0)
    def _zero():
        pool_ref[:] = jnp.zeros_like(pool_ref)

    pool_ref[:] += pool_blk


_pp_call = pl.pallas_call(
    _pp_body,
    grid=(_NB,),
    in_specs=[
        pl.BlockSpec((2, _BN, _D), lambda i: (0, i, 0)),
        pl.BlockSpec((_D, _D), lambda i: (0, 0)),
        pl.BlockSpec((1, _D), lambda i: (0, 0)),
        pl.BlockSpec((1, _D), lambda i: (0, 0)),
        pl.BlockSpec((1, _D), lambda i: (0, 0)),
        pl.BlockSpec((_G, _D), lambda i: (0, 0)),
        pl.BlockSpec((1, 1, _BN), lambda i: (i, 0, 0)),
    ],
    out_specs=[
        pl.BlockSpec((_BN, _D), lambda i: (i, 0)),
        pl.BlockSpec((_G, _D), lambda i: (0, 0)),
    ],
    out_shape=[
        jax.ShapeDtypeStruct((_N, _D), _F32),
        jax.ShapeDtypeStruct((_G, _D), _F32),
    ],
)


def _final_body(part_ref, wl_ref, bl_ref, lg_ref, lb_ref, wo_ref, bo_ref,
                out_ref):
    aggb = part_ref[0] + part_ref[1]
    hl = _dot(aggb, wl_ref[:]) + bl_ref[:]
    mu = jnp.mean(hl, axis=-1, keepdims=True)
    var = jnp.mean((hl - mu) ** 2, axis=-1, keepdims=True)
    hl = lg_ref[:] * (hl - mu) / jnp.sqrt(var + 1e-5) + lb_ref[:]
    out_ref[:] = jnp.maximum(_dot(hl, wo_ref[:]) + bo_ref[:], 0.0)


_final_call = pl.pallas_call(
    _final_body,
    grid=(_NB,),
    in_specs=[
        pl.BlockSpec((2, _BN, _D), lambda i: (0, i, 0)),
        pl.BlockSpec((_D, _D), lambda i: (0, 0)),
        pl.BlockSpec((1, _D), lambda i: (0, 0)),
        pl.BlockSpec((1, _D), lambda i: (0, 0)),
        pl.BlockSpec((1, _D), lambda i: (0, 0)),
        pl.BlockSpec((_D, _D), lambda i: (0, 0)),
        pl.BlockSpec((1, _D), lambda i: (0, 0)),
    ],
    out_specs=pl.BlockSpec((_BN, _D), lambda i: (i, 0)),
    out_shape=jax.ShapeDtypeStruct((_N, _D), _F32),
)


def _make_mp():
    scratch = ([pltpu.VMEM((_C,), jnp.int32)] * _NBUF
               + [pltpu.VMEM((_C,), jnp.int32)] * _NBUF
               + [pltpu.VMEM((_C, _D), _F32)] * _NBUF
               + [pltpu.VMEM((_C, _D), _F32)] * _NBUF
               + [pltpu.VMEM_SHARED((_N_PAD, _D), _F32)]
               + [pltpu.SemaphoreType.DMA] * (3 * _NBUF))

    @functools.partial(
        pl.kernel,
        out_type=jax.ShapeDtypeStruct((2 * _N, _D), _F32),
        mesh=plsc.VectorSubcoreMesh(core_axis_name="c", subcore_axis_name="s"),
        scratch_types=scratch,
    )
    def mp(hin_hbm, em_hbm, src_hbm, dst_hbm, zeros_hbm, out_hbm, *sc):
        sidx = sc[0:_NBUF]
        didx = sc[_NBUF:2 * _NBUF]
        rows = sc[2 * _NBUF:3 * _NBUF]
        emb = sc[3 * _NBUF:4 * _NBUF]
        agg = sc[4 * _NBUF]
        semA = sc[4 * _NBUF + 1:4 * _NBUF + 1 + _NBUF]
        semG = sc[4 * _NBUF + 1 + _NBUF:4 * _NBUF + 1 + 2 * _NBUF]
        semS = sc[4 * _NBUF + 1 + 2 * _NBUF:4 * _NBUF + 1 + 3 * _NBUF]

        c = lax.axis_index("c")
        s = lax.axis_index("s")
        nch = jnp.where(c == 0, _T0, _T1)
        cbase = c * _NS * _T0 + s * nch

        def startA(g, j):
            off = (cbase + g) * _C
            pltpu.async_copy(src_hbm.at[pl.ds(off, _C)], sidx[j], semA[j])
            pltpu.async_copy(dst_hbm.at[pl.ds(off, _C)], didx[j], semA[j])
            pltpu.async_copy(em_hbm.at[pl.ds(off, _C), :], emb[j], semA[j])

        def waitA(j):
            pltpu.make_async_copy(src_hbm.at[pl.ds(0, _C)], sidx[j],
                                  semA[j]).wait()
            pltpu.make_async_copy(dst_hbm.at[pl.ds(0, _C)], didx[j],
                                  semA[j]).wait()
            pltpu.make_async_copy(em_hbm.at[pl.ds(0, _C), :], emb[j],
                                  semA[j]).wait()

        def startG(j):
            pltpu.async_copy(hin_hbm.at[sidx[j]], rows[j], semG[j])

        def waitG(j):
            pltpu.make_async_copy(hin_hbm.at[sidx[j]], rows[j], semG[j]).wait()

        def compute(j):
            r, m = rows[j], emb[j]

            def edge(e, c2):
                for q in range(8):
                    sl = pl.ds(q * 16, 16)
                    r[e, sl] = jnp.maximum(r[e, sl] + m[e, sl], 0.0)
                return c2

            lax.fori_loop(0, _C, edge, 0)

        def startS(j):
            pltpu.async_copy(rows[j], agg.at[didx[j]], semS[j], add=True)

        def waitS(j):
            pltpu.make_async_copy(rows[j], agg.at[didx[j]], semS[j]).wait()

        pltpu.sync_copy(zeros_hbm, agg.at[pl.ds(s * (_N_PAD // _NS),
                                                _N_PAD // _NS)])
        plsc.subcore_barrier()

        startA(0, 0)
        startA(1, 1)
        waitA(0)
        startG(0)

        def quad(p, carry):
            for j in range(_NBUF):
                g = _NBUF * p + j
                j1 = (j + 1) % _NBUF
                j2 = (j + 2) % _NBUF

                @pl.when(g >= 2)
                def _ws(j2=j2):
                    waitS(j2)

                @pl.when(g + 2 < nch)
                def _sa(g=g, j2=j2):
                    startA(g + 2, j2)

                @pl.when(g + 1 < nch)
                def _sg(j1=j1):
                    waitA(j1)
                    startG(j1)

                waitG(j)
                compute(j)
                startS(j)
            return carry

        lax.fori_loop(0, nch // _NBUF, quad, 0)
        waitS(_NBUF - 2)
        waitS(_NBUF - 1)

        plsc.subcore_barrier()

        @pl.when(s < _NS - 1)
        def _wb_body():
            pltpu.sync_copy(agg.at[pl.ds(s * 624, 624)],
                            out_hbm.at[pl.ds(c * _N + s * 624, 624), :])

        @pl.when(s == _NS - 1)
        def _wb_tail():
            pltpu.sync_copy(agg.at[pl.ds((_NS - 1) * 624, 640)],
                            out_hbm.at[pl.ds(c * _N + (_NS - 1) * 624, 640), :])

    return mp


_mp_call = _make_mp()


def kernel(x, edge_attr, W0, b0, We, Wl, bl, ln_g, ln_b, vemb, vW1, vb1,
           vg1, vbt1, vW2, vb2, vg2, vbt2, Wo, bo, edge_index, batch):
    src = edge_index[0]
    dst = edge_index[1]
    pad = _E_PAD - _E
    src_p = jnp.concatenate([src, jnp.zeros((pad,), jnp.int32)])
    dst_p = jnp.concatenate([dst, jnp.full((pad,), _N, jnp.int32)])
    ea_p = jnp.concatenate([edge_attr, jnp.zeros((pad, _DE), _F32)], axis=0)
    batch3 = batch.reshape(_NB, 1, _BN)
    b0r = b0.reshape(1, _D)

    zeros2d = jnp.zeros((_N_PAD // _NS, _D), _F32)

    hin, pool = _init_call(x, W0, b0r, vemb, batch3)
    v = jnp.broadcast_to(vemb, (_G, _D))
    out = None
    for l in range(_L):
        em_l = _em_call(ea_p, We[l])
        part = _mp_call(hin, em_l, src_p, dst_p, zeros2d).reshape(2, _N, _D)
        if l < _L - 1:
            v = _vn_call(pool, v, vW1[l], vb1[l].reshape(1, _H2),
                         vg1[l].reshape(1, _H2), vbt1[l].reshape(1, _H2),
                         vW2[l], vb2[l].reshape(1, _D), vg2[l].reshape(1, _D),
                         vbt2[l].reshape(1, _D))
            hin, pool = _pp_call(part, Wl[l], bl[l].reshape(1, _D),
                                 ln_g[l].reshape(1, _D), ln_b[l].reshape(1, _D),
                                 v, batch3)
        else:
            out = _final_call(part, Wl[l], bl[l].reshape(1, _D),
                              ln_g[l].reshape(1, _D), ln_b[l].reshape(1, _D),
                              Wo, bo.reshape(1, _D))
    return out


# speedup vs baseline: 3.7955x; 1.3354x over previous
import functools

import jax
import jax.numpy as jnp
from jax import lax
from jax.experimental import pallas as pl
from jax.experimental.pallas import tpu as pltpu
from jax.experimental.pallas import tpu_sc as plsc

_N = 10000
_E = 320000
_D = 128
_DE = 16
_G = 64
_L = 3
_H2 = 256

_NC = 2
_NS = 16
_NW = _NC * _NS
_C = 32
_NBUF = 6
_T0 = 390
_T1 = 240
_NCHUNK = _T0 + _T1
_E_PAD = _NS * _NCHUNK * _C
_N_PAD = 10032
_RPT = _N // _NS

_BN = 2000
_NB = _N // _BN
_BE = _E_PAD // 32

_HI = lax.Precision.HIGHEST
_F32 = jnp.float32


def _dot(a, b):
    return jnp.dot(a, b, precision=_HI, preferred_element_type=_F32)


def _em_body(ea_ref, we_ref, out_ref):
    out_ref[:] = _dot(ea_ref[:], we_ref[:])


_em_call = pl.pallas_call(
    _em_body,
    grid=(_E_PAD // _BE,),
    in_specs=[
        pl.BlockSpec((_BE, _DE), lambda i: (i, 0)),
        pl.BlockSpec((_DE, _D), lambda i: (0, 0)),
    ],
    out_specs=pl.BlockSpec((_BE, _D), lambda i: (i, 0)),
    out_shape=jax.ShapeDtypeStruct((_E_PAD, _D), _F32),
)


def _onehot_t(batch_ref):
    bb = batch_ref[0]
    return (jnp.broadcast_to(bb, (_G, _BN))
            == lax.broadcasted_iota(jnp.int32, (_G, _BN), 0)).astype(_F32)


def _init_body(x_ref, w0_ref, b0_ref, vemb_ref, batch_ref, hin_ref, pool_ref):
    h = _dot(x_ref[:], w0_ref[:]) + b0_ref[:] + vemb_ref[:]
    hin_ref[:] = h
    oht = _onehot_t(batch_ref)
    pool_blk = lax.dot_general(oht, h, (((1,), (0,)), ((), ())),
                               precision=_HI, preferred_element_type=_F32)

    @pl.when(pl.program_id(0) == 0)
    def _zero():
        pool_ref[:] = jnp.zeros_like(pool_ref)

    pool_ref[:] += pool_blk


_init_call = pl.pallas_call(
    _init_body,
    grid=(_NB,),
    in_specs=[
        pl.BlockSpec((_BN, _D), lambda i: (i, 0)),
        pl.BlockSpec((_D, _D), lambda i: (0, 0)),
        pl.BlockSpec((1, _D), lambda i: (0, 0)),
        pl.BlockSpec((1, _D), lambda i: (0, 0)),
        pl.BlockSpec((1, 1, _BN), lambda i: (i, 0, 0)),
    ],
    out_specs=[
        pl.BlockSpec((_BN, _D), lambda i: (i, 0)),
        pl.BlockSpec((_G, _D), lambda i: (0, 0)),
    ],
    out_shape=[
        jax.ShapeDtypeStruct((_N, _D), _F32),
        jax.ShapeDtypeStruct((_G, _D), _F32),
    ],
)


def _vn_body(pool_ref, v_ref, w1_ref, b1_ref, g1_ref, t1_ref,
             w2_ref, b2_ref, g2_ref, t2_ref, out_ref):
    vt = pool_ref[:] + v_ref[:]
    z = _dot(vt, w1_ref[:]) + b1_ref[:]
    mu = jnp.mean(z, axis=0, keepdims=True)
    var = jnp.mean((z - mu) ** 2, axis=0, keepdims=True)
    z = jnp.maximum(g1_ref[:] * (z - mu) / jnp.sqrt(var + 1e-5) + t1_ref[:], 0.0)
    z2 = _dot(z, w2_ref[:]) + b2_ref[:]
    mu2 = jnp.mean(z2, axis=0, keepdims=True)
    var2 = jnp.mean((z2 - mu2) ** 2, axis=0, keepdims=True)
    out_ref[:] = jnp.maximum(
        g2_ref[:] * (z2 - mu2) / jnp.sqrt(var2 + 1e-5) + t2_ref[:], 0.0)


_vn_call = pl.pallas_call(
    _vn_body,
    out_shape=jax.ShapeDtypeStruct((_G, _D), _F32),
)


def _pp_body(part_ref, wl_ref, bl_ref, lg_ref, lb_ref, v_ref, batch_ref,
             hin_ref, pool_ref):
    aggb = part_ref[0] + part_ref[1]
    hl = _dot(aggb, wl_ref[:]) + bl_ref[:]
    mu = jnp.mean(hl, axis=-1, keepdims=True)
    var = jnp.mean((hl - mu) ** 2, axis=-1, keepdims=True)
    hl = lg_ref[:] * (hl - mu) / jnp.sqrt(var + 1e-5) + lb_ref[:]
    oht = _onehot_t(batch_ref)
    hin = hl + lax.dot_general(oht, v_ref[:], (((0,), (0,)), ((), ())),
                               precision=_HI, preferred_element_type=_F32)
    hin_ref[:] = hin
    pool_blk = lax.dot_general(oht, hin, (((1,), (0,)), ((), ())),
                               precision=_HI, preferred_element_type=_F32)

    @pl.when(pl.program_id(0) == 0)
    def _zero():
        pool_ref[:] = jnp.zeros_like(pool_ref)

    pool_ref[:] += pool_blk


_pp_call = pl.pallas_call(
    _pp_body,
    grid=(_NB,),
    in_specs=[
        pl.BlockSpec((2, _BN, _D), lambda i: (0, i, 0)),
        pl.BlockSpec((_D, _D), lambda i: (0, 0)),
        pl.BlockSpec((1, _D), lambda i: (0, 0)),
        pl.BlockSpec((1, _D), lambda i: (0, 0)),
        pl.BlockSpec((1, _D), lambda i: (0, 0)),
        pl.BlockSpec((_G, _D), lambda i: (0, 0)),
        pl.BlockSpec((1, 1, _BN), lambda i: (i, 0, 0)),
    ],
    out_specs=[
        pl.BlockSpec((_BN, _D), lambda i: (i, 0)),
        pl.BlockSpec((_G, _D), lambda i: (0, 0)),
    ],
    out_shape=[
        jax.ShapeDtypeStruct((_N, _D), _F32),
        jax.ShapeDtypeStruct((_G, _D), _F32),
    ],
)


def _final_body(part_ref, wl_ref, bl_ref, lg_ref, lb_ref, wo_ref, bo_ref,
                out_ref):
    aggb = part_ref[0] + part_ref[1]
    hl = _dot(aggb, wl_ref[:]) + bl_ref[:]
    mu = jnp.mean(hl, axis=-1, keepdims=True)
    var = jnp.mean((hl - mu) ** 2, axis=-1, keepdims=True)
    hl = lg_ref[:] * (hl - mu) / jnp.sqrt(var + 1e-5) + lb_ref[:]
    out_ref[:] = jnp.maximum(_dot(hl, wo_ref[:]) + bo_ref[:], 0.0)


_final_call = pl.pallas_call(
    _final_body,
    grid=(_NB,),
    in_specs=[
        pl.BlockSpec((2, _BN, _D), lambda i: (0, i, 0)),
        pl.BlockSpec((_D, _D), lambda i: (0, 0)),
        pl.BlockSpec((1, _D), lambda i: (0, 0)),
        pl.BlockSpec((1, _D), lambda i: (0, 0)),
        pl.BlockSpec((1, _D), lambda i: (0, 0)),
        pl.BlockSpec((_D, _D), lambda i: (0, 0)),
        pl.BlockSpec((1, _D), lambda i: (0, 0)),
    ],
    out_specs=pl.BlockSpec((_BN, _D), lambda i: (i, 0)),
    out_shape=jax.ShapeDtypeStruct((_N, _D), _F32),
)


def _make_mp():
    scratch = ([pltpu.VMEM((_C,), jnp.int32)] * _NBUF
               + [pltpu.VMEM((_C,), jnp.int32)] * _NBUF
               + [pltpu.VMEM((_C, _D), _F32)] * _NBUF
               + [pltpu.VMEM((_C, _D), _F32)] * _NBUF
               + [pltpu.VMEM_SHARED((_N_PAD, _D), _F32)]
               + [pltpu.SemaphoreType.DMA] * (3 * _NBUF))

    @functools.partial(
        pl.kernel,
        out_type=jax.ShapeDtypeStruct((2 * _N, _D), _F32),
        mesh=plsc.VectorSubcoreMesh(core_axis_name="c", subcore_axis_name="s"),
        scratch_types=scratch,
    )
    def mp(hin_hbm, em_hbm, src_hbm, dst_hbm, zeros_hbm, out_hbm, *sc):
        sidx = sc[0:_NBUF]
        didx = sc[_NBUF:2 * _NBUF]
        rows = sc[2 * _NBUF:3 * _NBUF]
        emb = sc[3 * _NBUF:4 * _NBUF]
        agg = sc[4 * _NBUF]
        semA = sc[4 * _NBUF + 1:4 * _NBUF + 1 + _NBUF]
        semG = sc[4 * _NBUF + 1 + _NBUF:4 * _NBUF + 1 + 2 * _NBUF]
        semS = sc[4 * _NBUF + 1 + 2 * _NBUF:4 * _NBUF + 1 + 3 * _NBUF]

        c = lax.axis_index("c")
        s = lax.axis_index("s")
        nch = jnp.where(c == 0, _T0, _T1)
        cbase = c * _NS * _T0 + s * nch

        def startA(g, j):
            off = (cbase + g) * _C
            pltpu.async_copy(src_hbm.at[pl.ds(off, _C)], sidx[j], semA[j])
            pltpu.async_copy(dst_hbm.at[pl.ds(off, _C)], didx[j], semA[j])
            pltpu.async_copy(em_hbm.at[pl.ds(off, _C), :], emb[j], semA[j])

        def waitA(j):
            pltpu.make_async_copy(src_hbm.at[pl.ds(0, _C)], sidx[j],
                                  semA[j]).wait()
            pltpu.make_async_copy(dst_hbm.at[pl.ds(0, _C)], didx[j],
                                  semA[j]).wait()
            pltpu.make_async_copy(em_hbm.at[pl.ds(0, _C), :], emb[j],
                                  semA[j]).wait()

        def startG(j):
            pltpu.async_copy(hin_hbm.at[sidx[j]], rows[j], semG[j])

        def waitG(j):
            pltpu.make_async_copy(hin_hbm.at[sidx[j]], rows[j], semG[j]).wait()

        def compute(j):
            r, m = rows[j], emb[j]

            def edge(e, c2):
                for q in range(8):
                    sl = pl.ds(q * 16, 16)
                    r[e, sl] = jnp.maximum(r[e, sl] + m[e, sl], 0.0)
                return c2

            lax.fori_loop(0, _C, edge, 0)

        def startS(j):
            pltpu.async_copy(rows[j], agg.at[didx[j]], semS[j], add=True)

        def waitS(j):
            pltpu.make_async_copy(rows[j], agg.at[didx[j]], semS[j]).wait()

        pltpu.sync_copy(zeros_hbm, agg.at[pl.ds(s * (_N_PAD // _NS),
                                                _N_PAD // _NS)])
        plsc.subcore_barrier()

        for j in range(4):
            startA(j, j)
        waitA(0)
        startG(0)
        waitA(1)
        startG(1)

        def sext(p, carry):
            for j in range(_NBUF):
                g = _NBUF * p + j
                j2 = (j + 2) % _NBUF
                j4 = (j + 4) % _NBUF
                j5 = (j + 5) % _NBUF

                @pl.when(g >= 1)
                def _ws(j5=j5):
                    waitS(j5)

                @pl.when(g + 4 < nch)
                def _sa(g=g, j4=j4):
                    startA(g + 4, j4)

                @pl.when(g + 2 < nch)
                def _sg(j2=j2):
                    waitA(j2)
                    startG(j2)

                waitG(j)
                compute(j)
                startS(j)
            return carry

        lax.fori_loop(0, nch // _NBUF, sext, 0)
        waitS(_NBUF - 1)

        plsc.subcore_barrier()

        @pl.when(s < _NS - 1)
        def _wb_body():
            pltpu.sync_copy(agg.at[pl.ds(s * 624, 624)],
                            out_hbm.at[pl.ds(c * _N + s * 624, 624), :])

        @pl.when(s == _NS - 1)
        def _wb_tail():
            pltpu.sync_copy(agg.at[pl.ds((_NS - 1) * 624, 640)],
                            out_hbm.at[pl.ds(c * _N + (_NS - 1) * 624, 640), :])

    return mp


_mp_call = _make_mp()


def kernel(x, edge_attr, W0, b0, We, Wl, bl, ln_g, ln_b, vemb, vW1, vb1,
           vg1, vbt1, vW2, vb2, vg2, vbt2, Wo, bo, edge_index, batch):
    src = edge_index[0]
    dst = edge_index[1]
    pad = _E_PAD - _E
    src_p = jnp.concatenate([src, jnp.zeros((pad,), jnp.int32)])
    dst_p = jnp.concatenate([dst, jnp.full((pad,), _N, jnp.int32)])
    ea_p = jnp.concatenate([edge_attr, jnp.zeros((pad, _DE), _F32)], axis=0)
    batch3 = batch.reshape(_NB, 1, _BN)
    b0r = b0.reshape(1, _D)

    zeros2d = jnp.zeros((_N_PAD // _NS, _D), _F32)

    hin, pool = _init_call(x, W0, b0r, vemb, batch3)
    v = jnp.broadcast_to(vemb, (_G, _D))
    out = None
    for l in range(_L):
        em_l = _em_call(ea_p, We[l])
        part = _mp_call(hin, em_l, src_p, dst_p, zeros2d).reshape(2, _N, _D)
        if l < _L - 1:
            v = _vn_call(pool, v, vW1[l], vb1[l].reshape(1, _H2),
                         vg1[l].reshape(1, _H2), vbt1[l].reshape(1, _H2),
                         vW2[l], vb2[l].reshape(1, _D), vg2[l].reshape(1, _D),
                         vbt2[l].reshape(1, _D))
            hin, pool = _pp_call(part, Wl[l], bl[l].reshape(1, _D),
                                 ln_g[l].reshape(1, _D), ln_b[l].reshape(1, _D),
                                 v, batch3)
        else:
            out = _final_call(part, Wl[l], bl[l].reshape(1, _D),
                              ln_g[l].reshape(1, _D), ln_b[l].reshape(1, _D),
                              Wo, bo.reshape(1, _D))
    return out
